# baseline (speedup 1.0000x reference)
import functools

import jax
import jax.numpy as jnp
from jax import lax
from jax.experimental import pallas as pl
from jax.experimental.pallas import tpu as pltpu
from jax.experimental.pallas import tpu_sc as plsc

N_TOKENS = 32768
D_MODEL = 768
NUM_EXPERTS = 64
BLK = 4096

_SC_INFO = plsc.get_sparse_core_info()
_NC, _NS, _NL = _SC_INFO.num_cores, _SC_INFO.num_subcores, _SC_INFO.num_lanes
_NW = _NC * _NS
_TPW = N_TOKENS // _NW


def _softmax_body(x_ref, w_ref, b_ref, soft_ref):
    x = x_ref[...]
    w = w_ref[...]
    logits = jax.lax.dot_general(
        x, w, (((1,), (0,)), ((), ())), preferred_element_type=jnp.float32)
    logits = logits + b_ref[...]
    m = jnp.max(logits, axis=-1, keepdims=True)
    e = jnp.exp(logits - m)
    s = jnp.sum(e, axis=-1, keepdims=True)
    soft_ref[...] = e / s


def _tc_softmax(x, W, b):
    n = x.shape[0]
    return pl.pallas_call(
        _softmax_body,
        grid=(n // BLK,),
        in_specs=[
            pl.BlockSpec((BLK, D_MODEL), lambda i: (i, 0)),
            pl.BlockSpec((D_MODEL, NUM_EXPERTS), lambda i: (0, 0)),
            pl.BlockSpec((1, NUM_EXPERTS), lambda i: (0, 0)),
        ],
        out_specs=pl.BlockSpec((BLK, NUM_EXPERTS), lambda i: (i, 0)),
        out_shape=jax.ShapeDtypeStruct((n, NUM_EXPERTS), jnp.float32),
    )(x, W, b.reshape(1, NUM_EXPERTS))


@functools.partial(
    pl.kernel,
    out_type=[
        jax.ShapeDtypeStruct((N_TOKENS * 2,), jnp.float32),
        jax.ShapeDtypeStruct((N_TOKENS * 2,), jnp.int32),
    ],
    mesh=plsc.VectorSubcoreMesh(core_axis_name="c", subcore_axis_name="s"),
    compiler_params=pltpu.CompilerParams(needs_layout_passes=False),
    scratch_types=[
        pltpu.VMEM((_TPW * NUM_EXPERTS,), jnp.float32),
        pltpu.VMEM((_TPW * 2,), jnp.float32),
        pltpu.VMEM((_TPW * 2,), jnp.int32),
    ],
)
def _sc_top2(soft_hbm, w_hbm, ei_hbm, slab, wbuf, eibuf):
    wid = lax.axis_index("s") * _NC + lax.axis_index("c")
    base = wid * _TPW
    pltpu.sync_copy(soft_hbm.at[pl.ds(base * NUM_EXPERTS, _TPW * NUM_EXPERTS)],
                    slab)

    lane = lax.iota(jnp.int32, _NL)

    hi_mask = jnp.full((_NL,), ~0x3F, jnp.int32)

    def group(g, carry):
        t = g * _NL + lane
        k1 = jnp.zeros((_NL,), jnp.int32)
        k2 = jnp.zeros((_NL,), jnp.int32)
        flat = t * NUM_EXPERTS
        for e in range(NUM_EXPERTS):
            v = plsc.load_gather(slab, [flat + e])
            key = (plsc.bitcast(v, jnp.int32) & hi_mask) | (63 - e)
            k2 = jnp.maximum(k2, jnp.minimum(key, k1))
            k1 = jnp.maximum(k1, key)
        m1 = plsc.bitcast(k1 & hi_mask, jnp.float32)
        m2 = plsc.bitcast(k2 & hi_mask, jnp.float32)
        i1 = 63 - (k1 & 0x3F)
        i2 = 63 - (k2 & 0x3F)
        tot = m1 + m2
        two_t = t * 2
        plsc.store_scatter(wbuf, [two_t], m1 / tot)
        plsc.store_scatter(wbuf, [two_t + 1], m2 / tot)
        plsc.store_scatter(eibuf, [two_t], i1)
        plsc.store_scatter(eibuf, [two_t + 1], i2)
        return carry

    lax.fori_loop(0, _TPW // _NL, group, 0)
    pltpu.sync_copy(wbuf, w_hbm.at[pl.ds(base * 2, _TPW * 2)])
    pltpu.sync_copy(eibuf, ei_hbm.at[pl.ds(base * 2, _TPW * 2)])


@jax.jit
def kernel(x, W, b):
    soft = _tc_softmax(x, W, b)
    wtop_flat, idx_flat = _sc_top2(soft.reshape(-1))
    return (wtop_flat.reshape(N_TOKENS, 2), idx_flat.reshape(N_TOKENS, 2),
            soft)


# speedup vs baseline: 1.8510x; 1.8510x over previous
import functools

import jax
import jax.numpy as jnp
from jax.experimental import pallas as pl
from jax.experimental.pallas import tpu as pltpu

N_TOKENS = 32768
D_MODEL = 768
NUM_EXPERTS = 64
BLK = 4096


def _router_body(x_ref, w_ref, b_ref, wtop_ref, idx_ref, soft_ref):
    x = x_ref[...]
    w = w_ref[...]
    logits = jax.lax.dot_general(
        x, w, (((1,), (0,)), ((), ())), preferred_element_type=jnp.float32)
    logits = logits + b_ref[...]
    m = jnp.max(logits, axis=-1, keepdims=True)
    e = jnp.exp(logits - m)
    s = jnp.sum(e, axis=-1, keepdims=True)
    soft_ref[...] = e / s

    lane = jax.lax.broadcasted_iota(jnp.int32, e.shape, 1)
    keys = ((jax.lax.bitcast_convert_type(e, jnp.int32) & ~0x3F)
            | (NUM_EXPERTS - 1 - lane))
    k1 = jnp.max(keys, axis=-1, keepdims=True)
    k2 = jnp.max(jnp.where(keys == k1, 0, keys), axis=-1, keepdims=True)
    e1 = jax.lax.bitcast_convert_type(k1 & ~0x3F, jnp.float32)
    e2 = jax.lax.bitcast_convert_type(k2 & ~0x3F, jnp.float32)
    i1 = (NUM_EXPERTS - 1) - (k1 & 0x3F)
    i2 = (NUM_EXPERTS - 1) - (k2 & 0x3F)
    tot = e1 + e2
    wtop_ref[...] = jnp.concatenate([e1 / tot, e2 / tot], axis=-1)
    idx_ref[...] = jnp.concatenate([i1, i2], axis=-1)


@jax.jit
def kernel(x, W, b):
    n = x.shape[0]
    grid = (n // BLK,)
    wtop, idx, soft = pl.pallas_call(
        _router_body,
        grid=grid,
        in_specs=[
            pl.BlockSpec((BLK, D_MODEL), lambda i: (i, 0)),
            pl.BlockSpec((D_MODEL, NUM_EXPERTS), lambda i: (0, 0)),
            pl.BlockSpec((1, NUM_EXPERTS), lambda i: (0, 0)),
        ],
        out_specs=[
            pl.BlockSpec((BLK, 2), lambda i: (i, 0)),
            pl.BlockSpec((BLK, 2), lambda i: (i, 0)),
            pl.BlockSpec((BLK, NUM_EXPERTS), lambda i: (i, 0)),
        ],
        out_shape=[
            jax.ShapeDtypeStruct((n, 2), jnp.float32),
            jax.ShapeDtypeStruct((n, 2), jnp.int32),
            jax.ShapeDtypeStruct((n, NUM_EXPERTS), jnp.float32),
        ],
    )(x, W, b.reshape(1, NUM_EXPERTS))
    return (wtop, idx, soft)


# speedup vs baseline: 1.8609x; 1.0054x over previous
import functools

import jax
import jax.numpy as jnp
from jax.experimental import pallas as pl
from jax.experimental.pallas import tpu as pltpu

N_TOKENS = 32768
D_MODEL = 768
NUM_EXPERTS = 64
BLK = 4096


def _router_body(x_ref, w_ref, b_ref, wtop_ref, idx_ref, soft_ref):
    x = x_ref[...]
    w = w_ref[...]
    logits = jax.lax.dot_general(
        x, w, (((1,), (0,)), ((), ())), preferred_element_type=jnp.float32)
    logits = logits + b_ref[...]
    m = jnp.max(logits, axis=-1, keepdims=True)
    e = jnp.exp(logits - m)
    s = jnp.sum(e, axis=-1, keepdims=True)
    soft = e / s
    soft_ref[...] = soft

    lane = jax.lax.broadcasted_iota(jnp.int32, soft.shape, 1)
    m1 = jnp.max(soft, axis=-1, keepdims=True)
    i1 = jnp.min(jnp.where(soft == m1, lane, NUM_EXPERTS), axis=-1,
                 keepdims=True)
    masked = jnp.where(lane == i1, -jnp.inf, soft)
    m2 = jnp.max(masked, axis=-1, keepdims=True)
    i2 = jnp.min(jnp.where(masked == m2, lane, NUM_EXPERTS), axis=-1,
                 keepdims=True)
    tot = m1 + m2
    wtop_ref[...] = jnp.concatenate([m1 / tot, m2 / tot], axis=-1)
    idx_ref[...] = jnp.concatenate([i1, i2], axis=-1)


@jax.jit
def kernel(x, W, b):
    n = x.shape[0]
    grid = (n // BLK,)
    wtop, idx, soft = pl.pallas_call(
        _router_body,
        grid=grid,
        in_specs=[
            pl.BlockSpec((BLK, D_MODEL), lambda i: (i, 0)),
            pl.BlockSpec((D_MODEL, NUM_EXPERTS), lambda i: (0, 0)),
            pl.BlockSpec((1, NUM_EXPERTS), lambda i: (0, 0)),
        ],
        out_specs=[
            pl.BlockSpec((BLK, 2), lambda i: (i, 0)),
            pl.BlockSpec((BLK, 2), lambda i: (i, 0)),
            pl.BlockSpec((BLK, NUM_EXPERTS), lambda i: (i, 0)),
        ],
        out_shape=[
            jax.ShapeDtypeStruct((n, 2), jnp.float32),
            jax.ShapeDtypeStruct((n, 2), jnp.int32),
            jax.ShapeDtypeStruct((n, NUM_EXPERTS), jnp.float32),
        ],
    )(x, W, b.reshape(1, NUM_EXPERTS))
    return (wtop, idx, soft)
